# Initial kernel scaffold; baseline (speedup 1.0000x reference)
import jax
import jax.numpy as jnp
from jax.experimental import pallas as pl


def kernel(x, edge_index, edge_weight, batch, W1, b1, W2, b2, W3, b3, Wl1, bl1, Wl2, bl2, Wl3, bl3, Wo, bo):
    raise NotImplementedError("write your pallas kernel here")


# speedup vs baseline: 5.3707x; 5.3707x over previous
import jax
import jax.numpy as jnp
from jax import lax
from jax.experimental import pallas as pl
from jax.experimental.pallas import tpu as pltpu
from jax.experimental.pallas import tpu_sc as plsc

N = 10000
D = 128
DH = D // 2
G = 64
NC = 2
NS = 16
NW = NC * NS
CK = 128
RPT = 624
TAIL = N - NS * RPT
ZR = N // NS
BLK = 2000
NBLK = N // BLK

_mesh = plsc.VectorSubcoreMesh(
    core_axis_name="c", subcore_axis_name="s", num_cores=NC, num_subcores=NS)


def _fill_zero_idx(idxz, sid):
    base = sid * ZR
    lanes = jnp.arange(16, dtype=jnp.int32)
    for r in range(idxz.shape[0]):
        for c in range(CK // 16):
            idxz[r, pl.ds(c * 16, 16)] = base + jnp.minimum(
                r * CK + c * 16 + lanes, ZR - 1)


def _zero_acc_indirect(zbuf, idxz, acc):
    def zb(r, _):
        for c in range(zbuf.shape[1] // 16):
            zbuf[r, pl.ds(c * 16, 16)] = jnp.zeros((16,), jnp.float32)
        return _
    lax.fori_loop(0, zbuf.shape[0], zb, None)
    for r in range(idxz.shape[0]):
        pltpu.sync_copy(zbuf, acc.at[idxz.at[r]])


def _copy_out(acc, out_hbm, cid, sid):
    base = pl.multiple_of(sid * RPT, 8)
    pltpu.sync_copy(acc.at[pl.ds(base, RPT)],
                    out_hbm.at[cid, pl.ds(base, RPT)])

    @pl.when(sid == NS - 1)
    def _():
        pltpu.sync_copy(acc.at[pl.ds(NS * RPT, TAIL)],
                        out_hbm.at[cid, pl.ds(NS * RPT, TAIL)])


def _deg_body(dst_hbm, wrows_hbm, out_hbm, dst_v, w16, zbuf, idxz, degacc):
    cid = lax.axis_index("c")
    sid = lax.axis_index("s")
    wid = sid * NC + cid
    nch = dst_v.shape[0]

    pltpu.sync_copy(dst_hbm.at[wid], dst_v)
    _fill_zero_idx(idxz, sid)
    _zero_acc_indirect(zbuf, idxz, degacc)
    plsc.subcore_barrier()

    def chunk(i, _):
        pltpu.sync_copy(wrows_hbm.at[wid, i], w16)
        pltpu.sync_copy(w16, degacc.at[dst_v.at[i]], add=True)
        return _
    lax.fori_loop(0, nch, chunk, None)
    plsc.subcore_barrier()
    _copy_out(degacc, out_hbm, cid, sid)


def _make_deg_call(nch):
    return pl.kernel(
        _deg_body,
        out_type=jax.ShapeDtypeStruct((NC, N, 16), jnp.float32),
        mesh=_mesh,
        scratch_types=[
            pltpu.VMEM((nch, CK), jnp.int32),
            pltpu.VMEM((CK, 16), jnp.float32),
            pltpu.VMEM((CK, 16), jnp.float32),
            pltpu.VMEM((5, CK), jnp.int32),
            pltpu.VMEM_SHARED((N, 16), jnp.float32),
        ],
    )


def _agg_body(y_hbm, src_hbm, dst_hbm, wrows_hbm, out_hbm,
              src_v, dst_v, w16, rows, zbuf, idxz, acc):
    cid = lax.axis_index("c")
    sid = lax.axis_index("s")
    wid = sid * NC + cid
    nch = src_v.shape[0]

    pltpu.sync_copy(src_hbm.at[wid], src_v)
    pltpu.sync_copy(dst_hbm.at[wid], dst_v)
    _fill_zero_idx(idxz, sid)
    _zero_acc_indirect(zbuf, idxz, acc)
    plsc.subcore_barrier()

    def chunk(i, _):
        pltpu.sync_copy(y_hbm.at[src_v.at[i]], rows)
        pltpu.sync_copy(wrows_hbm.at[wid, i], w16)

        def scale(j, _2):
            wb = w16[j, :]
            for c in range(DH // 16):
                rows[j, pl.ds(c * 16, 16)] = rows[j, pl.ds(c * 16, 16)] * wb
            return _2
        lax.fori_loop(0, CK, scale, None)

        pltpu.sync_copy(rows, acc.at[dst_v.at[i]], add=True)
        return _
    lax.fori_loop(0, nch, chunk, None)
    plsc.subcore_barrier()
    _copy_out(acc, out_hbm, cid, sid)


def _make_agg_call(nch):
    return pl.kernel(
        _agg_body,
        out_type=jax.ShapeDtypeStruct((NC, N, DH), jnp.float32),
        mesh=_mesh,
        compiler_params=pltpu.CompilerParams(use_tc_tiling_on_sc=False),
        scratch_types=[
            pltpu.VMEM((nch, CK), jnp.int32),
            pltpu.VMEM((nch, CK), jnp.int32),
            pltpu.VMEM((CK, 16), jnp.float32),
            pltpu.VMEM((CK, DH), jnp.float32),
            pltpu.VMEM((CK, DH), jnp.float32),
            pltpu.VMEM((5, CK), jnp.int32),
            pltpu.VMEM_SHARED((N, DH), jnp.float32),
        ],
    )


def _dis_of(degp_ref):
    d = degp_ref[0, :, 0:1] + degp_ref[1, :, 0:1] + 1.0
    return lax.rsqrt(d)


def _split_store(y, lo_ref, hi_ref):
    lo_ref[...] = y[:, :DH]
    hi_ref[...] = y[:, DH:]


def _combine(alo_ref, ahi_ref, ylo_ref, yhi_ref):
    lo = alo_ref[0] + alo_ref[1] + ylo_ref[...]
    hi = ahi_ref[0] + ahi_ref[1] + yhi_ref[...]
    return jnp.concatenate([lo, hi], axis=1)


def _pre_body(x_ref, degp_ref, w_ref, ylo_ref, yhi_ref):
    dis = _dis_of(degp_ref)
    y = jnp.dot(x_ref[...] * dis, w_ref[...],
                preferred_element_type=jnp.float32)
    _split_store(y, ylo_ref, yhi_ref)


def _mid_body(alo_ref, ahi_ref, ylo_ref, yhi_ref, degp_ref, b_ref, w_ref,
              olo_ref, ohi_ref):
    dis = _dis_of(degp_ref)
    s = _combine(alo_ref, ahi_ref, ylo_ref, yhi_ref) * dis + b_ref[...]
    h = jnp.tanh(s)
    y = jnp.dot(h * dis, w_ref[...], preferred_element_type=jnp.float32)
    _split_store(y, olo_ref, ohi_ref)


def _final_body(alo_ref, ahi_ref, ylo_ref, yhi_ref, degp_ref, b_ref,
                batch_ref, wl1_ref, bl1_ref, wl2_ref, bl2_ref, wl3_ref,
                bl3_ref, wo_ref, bo_ref, out_ref, sum_acc, cnt_acc):
    i = pl.program_id(0)
    dis = _dis_of(degp_ref)
    h3 = jnp.tanh(
        _combine(alo_ref, ahi_ref, ylo_ref, yhi_ref) * dis + b_ref[...])

    bb = batch_ref[0, 0, :]
    gids = lax.broadcasted_iota(jnp.int32, (G, BLK), 0)
    oh = (bb[None, :] == gids).astype(jnp.float32)

    @pl.when(i == 0)
    def _():
        sum_acc[...] = jnp.zeros((G, D), jnp.float32)
        cnt_acc[...] = jnp.zeros((G, D), jnp.float32)

    sum_acc[...] += jnp.dot(oh, h3, preferred_element_type=jnp.float32)
    cnt_acc[...] += jnp.sum(oh, axis=1)[:, None]

    @pl.when(i == NBLK - 1)
    def _():
        hp = sum_acc[...] / jnp.maximum(cnt_acc[...], 1.0)
        h = jnp.tanh(jnp.dot(hp, wl1_ref[...],
                             preferred_element_type=jnp.float32) + bl1_ref[...])
        h = jnp.tanh(jnp.dot(h, wl2_ref[...],
                             preferred_element_type=jnp.float32) + bl2_ref[...])
        h = jnp.tanh(jnp.dot(h, wl3_ref[...],
                             preferred_element_type=jnp.float32) + bl3_ref[...])
        o = jnp.dot(h, wo_ref[...],
                    preferred_element_type=jnp.float32) + bo_ref[...]
        out_ref[...] = o


def _row_spec(last):
    return pl.BlockSpec((BLK, last), lambda i: (i, 0))


def _full_spec(shape):
    return pl.BlockSpec(shape, lambda i: tuple(0 for _ in shape))


_agg_spec = pl.BlockSpec((NC, BLK, DH), lambda i: (0, i, 0))
_deg_spec = pl.BlockSpec((NC, BLK, 16), lambda i: (0, i, 0))
_yhalf = jax.ShapeDtypeStruct((N, DH), jnp.float32)

_pre_call = pl.pallas_call(
    _pre_body,
    grid=(NBLK,),
    in_specs=[_row_spec(D), _deg_spec, _full_spec((D, D))],
    out_specs=[_row_spec(DH), _row_spec(DH)],
    out_shape=[_yhalf, _yhalf],
)

_mid_call = pl.pallas_call(
    _mid_body,
    grid=(NBLK,),
    in_specs=[
        _agg_spec, _agg_spec, _row_spec(DH), _row_spec(DH), _deg_spec,
        _full_spec((1, D)), _full_spec((D, D)),
    ],
    out_specs=[_row_spec(DH), _row_spec(DH)],
    out_shape=[_yhalf, _yhalf],
)

_final_call = pl.pallas_call(
    _final_body,
    grid=(NBLK,),
    in_specs=[
        _agg_spec, _agg_spec, _row_spec(DH), _row_spec(DH), _deg_spec,
        _full_spec((1, D)),
        pl.BlockSpec((1, 1, BLK), lambda i: (i, 0, 0)),
        _full_spec((D, D)), _full_spec((1, D)),
        _full_spec((D, D)), _full_spec((1, D)),
        _full_spec((D, D)), _full_spec((1, D)),
        _full_spec((D, D)), _full_spec((1, D)),
    ],
    out_specs=pl.BlockSpec((G, D), lambda i: (0, 0)),
    out_shape=jax.ShapeDtypeStruct((G, D), jnp.float32),
    scratch_shapes=[
        pltpu.VMEM((G, D), jnp.float32),
        pltpu.VMEM((G, D), jnp.float32),
    ],
)


@jax.jit
def kernel(x, edge_index, edge_weight, batch,
           W1, b1, W2, b2, W3, b3,
           Wl1, bl1, Wl2, bl2, Wl3, bl3, Wo, bo):
    E = edge_index.shape[1]
    nch = -(-E // (NW * CK))
    epad = NW * nch * CK

    src = jnp.pad(edge_index[0].astype(jnp.int32), (0, epad - E))
    dst = jnp.pad(edge_index[1].astype(jnp.int32), (0, epad - E))
    w = jnp.pad(edge_weight.astype(jnp.float32), (0, epad - E))
    src_r = src.reshape(NW, nch, CK)
    dst_r = dst.reshape(NW, nch, CK)
    wrows = jnp.broadcast_to(w[:, None], (epad, 16)).reshape(NW, nch, CK, 16)
    batch_r = batch.astype(jnp.int32).reshape(NBLK, 1, BLK)

    deg_call = _make_deg_call(nch)
    agg_call = _make_agg_call(nch)

    degp = deg_call(dst_r, wrows)
    ylo, yhi = _pre_call(x, degp, W1)

    for _, (bias, Wn) in enumerate([(b1, W2), (b2, W3)]):
        alo = agg_call(ylo, src_r, dst_r, wrows)
        ahi = agg_call(yhi, src_r, dst_r, wrows)
        ylo, yhi = _mid_call(alo, ahi, ylo, yhi, degp,
                             bias.reshape(1, D), Wn)

    alo = agg_call(ylo, src_r, dst_r, wrows)
    ahi = agg_call(yhi, src_r, dst_r, wrows)
    out128 = _final_call(alo, ahi, ylo, yhi, degp, b3.reshape(1, D),
                         batch_r,
                         Wl1, bl1.reshape(1, D), Wl2, bl2.reshape(1, D),
                         Wl3, bl3.reshape(1, D),
                         jnp.pad(Wo, ((0, 0), (0, D - 1))),
                         jnp.broadcast_to(bo.reshape(1, 1), (1, D)))
    return out128[:, :1]


# speedup vs baseline: 5.3737x; 1.0006x over previous
import jax
import jax.numpy as jnp
from jax import lax
from jax.experimental import pallas as pl
from jax.experimental.pallas import tpu as pltpu
from jax.experimental.pallas import tpu_sc as plsc

N = 10000
D = 128
DH = D // 2
G = 64
NC = 2
NS = 16
NW = NC * NS
CK = 128
RPT = 624
TAIL = N - NS * RPT
ZR = N // NS
BLK = 2000
NBLK = N // BLK

_mesh = plsc.VectorSubcoreMesh(
    core_axis_name="c", subcore_axis_name="s", num_cores=NC, num_subcores=NS)


def _fill_zero_idx(idxz, sid):
    base = sid * ZR
    lanes = jnp.arange(16, dtype=jnp.int32)
    for r in range(idxz.shape[0]):
        for c in range(CK // 16):
            idxz[r, pl.ds(c * 16, 16)] = base + jnp.minimum(
                r * CK + c * 16 + lanes, ZR - 1)


def _zero_acc_indirect(zbuf, idxz, acc):
    def zb(r, _):
        for c in range(zbuf.shape[1] // 16):
            zbuf[r, pl.ds(c * 16, 16)] = jnp.zeros((16,), jnp.float32)
        return _
    lax.fori_loop(0, zbuf.shape[0], zb, None)
    for r in range(idxz.shape[0]):
        pltpu.sync_copy(zbuf, acc.at[idxz.at[r]])


def _copy_out(acc, out_hbm, cid, sid):
    base = pl.multiple_of(sid * RPT, 8)
    pltpu.sync_copy(acc.at[pl.ds(base, RPT)],
                    out_hbm.at[cid, pl.ds(base, RPT)])

    @pl.when(sid == NS - 1)
    def _():
        pltpu.sync_copy(acc.at[pl.ds(NS * RPT, TAIL)],
                        out_hbm.at[cid, pl.ds(NS * RPT, TAIL)])


def _deg_body(dst_hbm, wrows_hbm, out_hbm, dst_v, w16, zbuf, idxz, degacc):
    cid = lax.axis_index("c")
    sid = lax.axis_index("s")
    wid = sid * NC + cid
    nch = dst_v.shape[0]

    pltpu.sync_copy(dst_hbm.at[wid], dst_v)
    _fill_zero_idx(idxz, sid)
    _zero_acc_indirect(zbuf, idxz, degacc)
    plsc.subcore_barrier()

    def chunk(i, _):
        pltpu.sync_copy(wrows_hbm.at[wid, i], w16)
        pltpu.sync_copy(w16, degacc.at[dst_v.at[i]], add=True)
        return _
    lax.fori_loop(0, nch, chunk, None)
    plsc.subcore_barrier()
    _copy_out(degacc, out_hbm, cid, sid)


def _make_deg_call(nch):
    return pl.kernel(
        _deg_body,
        out_type=jax.ShapeDtypeStruct((NC, N, 16), jnp.float32),
        mesh=_mesh,
        scratch_types=[
            pltpu.VMEM((nch, CK), jnp.int32),
            pltpu.VMEM((CK, 16), jnp.float32),
            pltpu.VMEM((CK, 16), jnp.float32),
            pltpu.VMEM((5, CK), jnp.int32),
            pltpu.VMEM_SHARED((N, 16), jnp.float32),
        ],
    )


def _agg_body(y_hbm, src_hbm, dst_hbm, wrows_hbm, out_hbm,
              src_v, dst_v, w16, rows, zbuf, idxz, acc):
    cid = lax.axis_index("c")
    sid = lax.axis_index("s")
    wid = sid * NC + cid
    nch = src_v.shape[0]

    pltpu.sync_copy(src_hbm.at[wid], src_v)
    pltpu.sync_copy(dst_hbm.at[wid], dst_v)
    _fill_zero_idx(idxz, sid)
    _zero_acc_indirect(zbuf, idxz, acc)
    plsc.subcore_barrier()

    def chunk(i, _):
        pltpu.sync_copy(y_hbm.at[src_v.at[i]], rows)
        pltpu.sync_copy(wrows_hbm.at[wid, i], w16)

        def scale(j, _2):
            wb = w16[j, :]
            for c in range(DH // 16):
                rows[j, pl.ds(c * 16, 16)] = rows[j, pl.ds(c * 16, 16)] * wb
            return _2
        lax.fori_loop(0, CK, scale, None)

        pltpu.sync_copy(rows, acc.at[dst_v.at[i]], add=True)
        return _
    lax.fori_loop(0, nch, chunk, None)
    plsc.subcore_barrier()
    _copy_out(acc, out_hbm, cid, sid)


def _make_agg_call(nch):
    return pl.kernel(
        _agg_body,
        out_type=jax.ShapeDtypeStruct((NC, N, DH), jnp.float32),
        mesh=_mesh,
        compiler_params=pltpu.CompilerParams(use_tc_tiling_on_sc=False),
        scratch_types=[
            pltpu.VMEM((nch, CK), jnp.int32),
            pltpu.VMEM((nch, CK), jnp.int32),
            pltpu.VMEM((CK, 16), jnp.float32),
            pltpu.VMEM((CK, DH), jnp.float32),
            pltpu.VMEM((CK, DH), jnp.float32),
            pltpu.VMEM((5, CK), jnp.int32),
            pltpu.VMEM_SHARED((N, DH), jnp.float32),
        ],
    )


def _dis_of(degp_ref):
    d = degp_ref[0, :, 0:1] + degp_ref[1, :, 0:1] + 1.0
    return lax.rsqrt(d)


def _split_store(y, lo_ref, hi_ref):
    lo_ref[...] = y[:, :DH]
    hi_ref[...] = y[:, DH:]


def _combine(alo_ref, ahi_ref, ylo_ref, yhi_ref):
    lo = alo_ref[0] + alo_ref[1] + ylo_ref[...]
    hi = ahi_ref[0] + ahi_ref[1] + yhi_ref[...]
    return jnp.concatenate([lo, hi], axis=1)


def _pre_body(x_ref, degp_ref, w_ref, ylo_ref, yhi_ref):
    dis = _dis_of(degp_ref)
    y = jnp.dot(x_ref[...] * dis, w_ref[...],
                preferred_element_type=jnp.float32)
    _split_store(y, ylo_ref, yhi_ref)


def _mid_body(alo_ref, ahi_ref, ylo_ref, yhi_ref, degp_ref, b_ref, w_ref,
              olo_ref, ohi_ref):
    dis = _dis_of(degp_ref)
    s = _combine(alo_ref, ahi_ref, ylo_ref, yhi_ref) * dis + b_ref[...]
    h = jnp.tanh(s)
    y = jnp.dot(h * dis, w_ref[...], preferred_element_type=jnp.float32)
    _split_store(y, olo_ref, ohi_ref)


def _final_body(alo_ref, ahi_ref, ylo_ref, yhi_ref, degp_ref, b_ref,
                batch_ref, wl1_ref, bl1_ref, wl2_ref, bl2_ref, wl3_ref,
                bl3_ref, wo_ref, bo_ref, out_ref, sum_acc, cnt_acc):
    i = pl.program_id(0)
    dis = _dis_of(degp_ref)
    h3 = jnp.tanh(
        _combine(alo_ref, ahi_ref, ylo_ref, yhi_ref) * dis + b_ref[...])

    bb = batch_ref[0, 0, :]
    gids = lax.broadcasted_iota(jnp.int32, (G, BLK), 0)
    oh = (bb[None, :] == gids).astype(jnp.float32)

    @pl.when(i == 0)
    def _():
        sum_acc[...] = jnp.zeros((G, D), jnp.float32)
        cnt_acc[...] = jnp.zeros((G, D), jnp.float32)

    sum_acc[...] += jnp.dot(oh, h3, preferred_element_type=jnp.float32)
    cnt_acc[...] += jnp.sum(oh, axis=1)[:, None]

    @pl.when(i == NBLK - 1)
    def _():
        hp = sum_acc[...] / jnp.maximum(cnt_acc[...], 1.0)
        h = jnp.tanh(jnp.dot(hp, wl1_ref[...],
                             preferred_element_type=jnp.float32) + bl1_ref[...])
        h = jnp.tanh(jnp.dot(h, wl2_ref[...],
                             preferred_element_type=jnp.float32) + bl2_ref[...])
        h = jnp.tanh(jnp.dot(h, wl3_ref[...],
                             preferred_element_type=jnp.float32) + bl3_ref[...])
        o = jnp.dot(h, wo_ref[...],
                    preferred_element_type=jnp.float32) + bo_ref[...]
        out_ref[...] = o


def _row_spec(last):
    return pl.BlockSpec((BLK, last), lambda i: (i, 0))


def _full_spec(shape):
    return pl.BlockSpec(shape, lambda i: tuple(0 for _ in shape))


_agg_spec = pl.BlockSpec((NC, BLK, DH), lambda i: (0, i, 0))
_deg_spec = pl.BlockSpec((NC, BLK, 16), lambda i: (0, i, 0))
_yhalf = jax.ShapeDtypeStruct((N, DH), jnp.float32)

_pre_call = pl.pallas_call(
    _pre_body,
    grid=(NBLK,),
    in_specs=[_row_spec(D), _deg_spec, _full_spec((D, D))],
    out_specs=[_row_spec(DH), _row_spec(DH)],
    out_shape=[_yhalf, _yhalf],
)

_mid_call = pl.pallas_call(
    _mid_body,
    grid=(NBLK,),
    in_specs=[
        _agg_spec, _agg_spec, _row_spec(DH), _row_spec(DH), _deg_spec,
        _full_spec((1, D)), _full_spec((D, D)),
    ],
    out_specs=[_row_spec(DH), _row_spec(DH)],
    out_shape=[_yhalf, _yhalf],
)

_final_call = pl.pallas_call(
    _final_body,
    grid=(NBLK,),
    in_specs=[
        _agg_spec, _agg_spec, _row_spec(DH), _row_spec(DH), _deg_spec,
        _full_spec((1, D)),
        pl.BlockSpec((1, 1, BLK), lambda i: (i, 0, 0)),
        _full_spec((D, D)), _full_spec((1, D)),
        _full_spec((D, D)), _full_spec((1, D)),
        _full_spec((D, D)), _full_spec((1, D)),
        _full_spec((D, D)), _full_spec((1, D)),
    ],
    out_specs=pl.BlockSpec((G, D), lambda i: (0, 0)),
    out_shape=jax.ShapeDtypeStruct((G, D), jnp.float32),
    scratch_shapes=[
        pltpu.VMEM((G, D), jnp.float32),
        pltpu.VMEM((G, D), jnp.float32),
    ],
)


@jax.jit
def kernel(x, edge_index, edge_weight, batch,
           W1, b1, W2, b2, W3, b3,
           Wl1, bl1, Wl2, bl2, Wl3, bl3, Wo, bo):
    E = edge_index.shape[1]
    nch = -(-E // (NW * CK))
    epad = NW * nch * CK

    src = jnp.pad(edge_index[0].astype(jnp.int32), (0, epad - E))
    dst = jnp.pad(edge_index[1].astype(jnp.int32), (0, epad - E))
    w = jnp.pad(edge_weight.astype(jnp.float32), (0, epad - E))
    src_r = src.reshape(NW, nch, CK)
    dst_r = dst.reshape(NW, nch, CK)
    wrows = jnp.broadcast_to(w[:, None], (epad, 16)).reshape(NW, nch, CK, 16)
    batch_r = batch.astype(jnp.int32).reshape(NBLK, 1, BLK)

    deg_call = _make_deg_call(nch)
    agg_call = _make_agg_call(nch)

    degp = deg_call(dst_r, wrows)
    ylo, yhi = _pre_call(x, degp, W1)

    for bias, Wn in [(b1, W2), (b2, W3)]:
        alo = agg_call(ylo, src_r, dst_r, wrows)
        ahi = agg_call(yhi, src_r, dst_r, wrows)
        ylo, yhi = _mid_call(alo, ahi, ylo, yhi, degp,
                             bias.reshape(1, D), Wn)

    alo = agg_call(ylo, src_r, dst_r, wrows)
    ahi = agg_call(yhi, src_r, dst_r, wrows)
    out128 = _final_call(alo, ahi, ylo, yhi, degp, b3.reshape(1, D),
                         batch_r,
                         Wl1, bl1.reshape(1, D), Wl2, bl2.reshape(1, D),
                         Wl3, bl3.reshape(1, D),
                         jnp.pad(Wo, ((0, 0), (0, D - 1))),
                         jnp.broadcast_to(bo.reshape(1, 1), (1, D)))
    return out128[:, :1]
